# Initial kernel scaffold; baseline (speedup 1.0000x reference)
import jax
import jax.numpy as jnp
from jax.experimental import pallas as pl


def kernel(x, edge_index, emb, W_pre1, b_pre1, W_pre2, b_pre2, Wc1, bc1, Wc2, bc2, Wc3, bc3, W_post1, W_post2):
    raise NotImplementedError("write your pallas kernel here")


# speedup vs baseline: 15.7887x; 15.7887x over previous
import functools

import jax
import jax.numpy as jnp
from jax import lax
from jax.experimental import pallas as pl
from jax.experimental.pallas import tpu as pltpu
from jax.experimental.pallas import tpu_sc as plsc

N = 10000
D = 128
E = 320000
NC = 2
NS = 16
NW = NC * NS
EPW = E // NW
WIN = 80
NWIN = EPW // WIN
RPT_A = 624
RPT_B = N - 15 * RPT_A

_f32 = jnp.float32


def _leaky(v):
    return jnp.where(v >= 0, v, 0.01 * v)


_sc_mesh = plsc.VectorSubcoreMesh(core_axis_name="c", subcore_axis_name="s")


@functools.partial(
    pl.kernel,
    mesh=_sc_mesh,
    out_type=jax.ShapeDtypeStruct((NC, N), _f32),
    scratch_types=[
        pltpu.VMEM_SHARED((N,), _f32),
        pltpu.VMEM((NWIN, WIN), jnp.int32),
        pltpu.VMEM((WIN,), _f32),
    ],
)
def _sc_deg(dst_hbm, zero_hbm, dp_hbm, dacc, dstv, onesv):
    cid = lax.axis_index("c")
    sid = lax.axis_index("s")
    wid = sid * NC + cid
    pltpu.sync_copy(dst_hbm.at[wid], dstv)
    for k in range(WIN // 16):
        onesv[pl.ds(k * 16, 16)] = jnp.ones((16,), _f32)

    @pl.when(sid == 0)
    def _():
        pltpu.sync_copy(zero_hbm, dacc)

    plsc.subcore_barrier()

    def body(j, carry):
        pltpu.sync_copy(onesv, dacc.at[dstv.at[j]], add=True)
        return carry

    lax.fori_loop(0, NWIN, body, 0)
    plsc.subcore_barrier()

    @pl.when(sid == 0)
    def _():
        pltpu.sync_copy(dacc, dp_hbm.at[cid])


@functools.partial(
    pl.kernel,
    mesh=_sc_mesh,
    out_type=jax.ShapeDtypeStruct((NC, N, D), _f32),
    scratch_types=[
        pltpu.VMEM_SHARED((N, D), _f32),
        pltpu.VMEM((NWIN, WIN), jnp.int32),
        pltpu.VMEM((NWIN, WIN), jnp.int32),
        pltpu.VMEM((WIN, D), _f32),
        pltpu.SemaphoreType.DMA,
    ],
)
def _sc_agg(g_hbm, src_hbm, dst_hbm, zrows_hbm, a_hbm, acc, srcv, dstv, rowbuf, sem):
    cid = lax.axis_index("c")
    sid = lax.axis_index("s")
    wid = sid * NC + cid
    row0 = pl.multiple_of(sid * RPT_A, 8)
    pltpu.sync_copy(src_hbm.at[wid], srcv)
    pltpu.sync_copy(dst_hbm.at[wid], dstv)

    @pl.when((cid == 0) & (sid < 15))
    def _():
        pltpu.sync_copy(g_hbm.at[pl.ds(row0, RPT_A)], acc.at[pl.ds(row0, RPT_A)])

    @pl.when((cid == 0) & (sid == 15))
    def _():
        pltpu.sync_copy(g_hbm.at[pl.ds(15 * RPT_A, RPT_B)],
                        acc.at[pl.ds(15 * RPT_A, RPT_B)])

    @pl.when((cid == 1) & (sid < 15))
    def _():
        pltpu.sync_copy(zrows_hbm.at[pl.ds(0, RPT_A)], acc.at[pl.ds(row0, RPT_A)])

    @pl.when((cid == 1) & (sid == 15))
    def _():
        pltpu.sync_copy(zrows_hbm, acc.at[pl.ds(15 * RPT_A, RPT_B)])

    plsc.subcore_barrier()

    def body(j, carry):
        pltpu.async_copy(g_hbm.at[srcv.at[j]], rowbuf, sem).wait()
        pltpu.sync_copy(rowbuf, acc.at[dstv.at[j]], add=True)
        return carry

    lax.fori_loop(0, NWIN, body, 0)
    plsc.subcore_barrier()

    @pl.when(sid < 15)
    def _():
        pltpu.sync_copy(acc.at[pl.ds(row0, RPT_A)],
                        a_hbm.at[cid, pl.ds(row0, RPT_A)])

    @pl.when(sid == 15)
    def _():
        pltpu.sync_copy(acc.at[pl.ds(15 * RPT_A, RPT_B)],
                        a_hbm.at[cid, pl.ds(15 * RPT_A, RPT_B)])


def _tc_pre_body(x_ref, emb_ref, wp1_ref, bp1_ref, wp2_ref, bp2_ref,
                 wc1_ref, dpt_ref, h0_ref, dinv_ref, g1_ref):
    table = jnp.dot(emb_ref[...], wp1_ref[...],
                    preferred_element_type=_f32) + bp1_ref[...]
    table = jnp.dot(_leaky(table), wp2_ref[...],
                    preferred_element_type=_f32) + bp2_ref[...]
    onehot = (lax.broadcasted_iota(jnp.int32, (N, D), 1)
              == x_ref[...]).astype(_f32)
    h0 = jnp.dot(onehot, table, preferred_element_type=_f32)
    deg = dpt_ref[:, 0:1] + dpt_ref[:, 1:2] + 1.0
    dinv = lax.rsqrt(deg)
    h0_ref[...] = h0
    dinv_ref[...] = dinv
    g1_ref[...] = jnp.dot(h0, wc1_ref[...], preferred_element_type=_f32) * dinv


_tc_pre = pl.pallas_call(
    _tc_pre_body,
    out_shape=[
        jax.ShapeDtypeStruct((N, D), _f32),
        jax.ShapeDtypeStruct((N, 1), _f32),
        jax.ShapeDtypeStruct((N, D), _f32),
    ],
)


def _tc_mid_body(a_ref, hprev_ref, dinv_ref, bc_ref, wcn_ref,
                 hnew_ref, gnext_ref):
    s = a_ref[0] + a_ref[1]
    hnew = _leaky(dinv_ref[...] * s + bc_ref[...]) + hprev_ref[...]
    hnew_ref[...] = hnew
    gnext_ref[...] = jnp.dot(hnew, wcn_ref[...],
                             preferred_element_type=_f32) * dinv_ref[...]


_tc_mid = pl.pallas_call(
    _tc_mid_body,
    out_shape=[
        jax.ShapeDtypeStruct((N, D), _f32),
        jax.ShapeDtypeStruct((N, D), _f32),
    ],
)


def _tc_post_body(a_ref, hprev_ref, dinv_ref, bc3_ref, wpost1_ref,
                  wpost2_ref, pred_ref):
    s = a_ref[0] + a_ref[1]
    h3 = _leaky(dinv_ref[...] * s + bc3_ref[...]) + hprev_ref[...]
    p = _leaky(jnp.dot(h3, wpost1_ref[...], preferred_element_type=_f32))
    pred_ref[...] = jnp.dot(p, wpost2_ref[...], preferred_element_type=_f32)


_tc_post = pl.pallas_call(
    _tc_post_body,
    out_shape=jax.ShapeDtypeStruct((N, 1), _f32),
)


def kernel(x, edge_index, emb, W_pre1, b_pre1, W_pre2, b_pre2,
           Wc1, bc1, Wc2, bc2, Wc3, bc3, W_post1, W_post2):
    ei = edge_index.astype(jnp.int32)
    src3 = ei[0].reshape(NW, NWIN, WIN)
    dst3 = ei[1].reshape(NW, NWIN, WIN)
    x2 = x.astype(jnp.int32).reshape(N, 1)
    zero_n = jnp.zeros((N,), _f32)
    zrows = jnp.zeros((RPT_B, D), _f32)

    dp = _sc_deg(dst3, zero_n)
    dpt = dp.T
    h0, dinv, g = _tc_pre(x2, emb, W_pre1, b_pre1.reshape(1, D),
                          W_pre2, b_pre2.reshape(1, D), Wc1, dpt)
    h = h0
    for bc, wnext in ((bc1, Wc2), (bc2, Wc3)):
        a = _sc_agg(g, src3, dst3, zrows)
        h, g = _tc_mid(a, h, dinv, bc.reshape(1, D), wnext)
    a = _sc_agg(g, src3, dst3, zrows)
    return _tc_post(a, h, dinv, bc3.reshape(1, D), W_post1, W_post2)


# speedup vs baseline: 19.4091x; 1.2293x over previous
import functools

import jax
import jax.numpy as jnp
from jax import lax
from jax.experimental import pallas as pl
from jax.experimental.pallas import tpu as pltpu
from jax.experimental.pallas import tpu_sc as plsc

N = 10000
D = 128
E = 320000
NC = 2
NS = 16
NW = NC * NS
EPW = E // NW
WIN = 80
NWIN = EPW // WIN
NCHUNK = 5
CH = NWIN // NCHUNK
RPT_A = 624
RPT_B = N - 15 * RPT_A

_f32 = jnp.float32


def _leaky(v):
    return jnp.where(v >= 0, v, 0.01 * v)


_sc_mesh = plsc.VectorSubcoreMesh(core_axis_name="c", subcore_axis_name="s")


@functools.partial(
    pl.kernel,
    mesh=_sc_mesh,
    out_type=jax.ShapeDtypeStruct((NC, N), _f32),
    scratch_types=[
        pltpu.VMEM_SHARED((N,), _f32),
        pltpu.VMEM((NCHUNK, CH, WIN), jnp.int32),
        pltpu.VMEM((WIN,), _f32),
    ],
)
def _sc_deg(dst_hbm, zero_hbm, dp_hbm, dacc, dstv, onesv):
    cid = lax.axis_index("c")
    sid = lax.axis_index("s")
    wid = sid * NC + cid
    pltpu.sync_copy(dst_hbm.at[wid], dstv)
    for k in range(WIN // 16):
        onesv[pl.ds(k * 16, 16)] = jnp.ones((16,), _f32)

    @pl.when(sid == 0)
    def _():
        pltpu.sync_copy(zero_hbm, dacc)

    plsc.subcore_barrier()

    def chunk(c, carry):
        def body(j, carry2):
            pltpu.sync_copy(onesv, dacc.at[dstv.at[c, j]], add=True)
            return carry2

        lax.fori_loop(0, CH, body, 0)
        return carry

    lax.fori_loop(0, NCHUNK, chunk, 0)
    plsc.subcore_barrier()

    @pl.when(sid == 0)
    def _():
        pltpu.sync_copy(dacc, dp_hbm.at[cid])


@functools.partial(
    pl.kernel,
    mesh=_sc_mesh,
    out_type=jax.ShapeDtypeStruct((NC, N, D), _f32),
    scratch_types=[
        pltpu.VMEM_SHARED((N, D), _f32),
        pltpu.VMEM((CH, WIN), jnp.int32),
        pltpu.VMEM((CH, WIN), jnp.int32),
        pltpu.VMEM((WIN, D), _f32),
        pltpu.VMEM((WIN, D), _f32),
        pltpu.SemaphoreType.DMA,
        pltpu.SemaphoreType.DMA,
    ],
)
def _sc_agg(g_hbm, src_hbm, dst_hbm, zrows_hbm, a_hbm, acc, srcv, dstv,
            rowbuf0, rowbuf1, sem0, sem1):
    cid = lax.axis_index("c")
    sid = lax.axis_index("s")
    wid = sid * NC + cid
    row0 = pl.multiple_of(sid * RPT_A, 8)

    @pl.when((cid == 0) & (sid < 15))
    def _():
        pltpu.sync_copy(g_hbm.at[pl.ds(row0, RPT_A)], acc.at[pl.ds(row0, RPT_A)])

    @pl.when((cid == 0) & (sid == 15))
    def _():
        pltpu.sync_copy(g_hbm.at[pl.ds(15 * RPT_A, RPT_B)],
                        acc.at[pl.ds(15 * RPT_A, RPT_B)])

    @pl.when((cid == 1) & (sid < 15))
    def _():
        pltpu.sync_copy(zrows_hbm.at[pl.ds(0, RPT_A)], acc.at[pl.ds(row0, RPT_A)])

    @pl.when((cid == 1) & (sid == 15))
    def _():
        pltpu.sync_copy(zrows_hbm, acc.at[pl.ds(15 * RPT_A, RPT_B)])

    plsc.subcore_barrier()

    def chunk(c, carry):
        pltpu.sync_copy(src_hbm.at[wid, c], srcv)
        pltpu.sync_copy(dst_hbm.at[wid, c], dstv)
        pltpu.async_copy(g_hbm.at[srcv.at[0]], rowbuf0, sem0)

        def body(j, carry2):
            def one(buf_cur, sem_cur, buf_nxt, sem_nxt):
                pltpu.make_async_copy(g_hbm.at[pl.ds(0, WIN)], buf_cur,
                                      sem_cur).wait()

                @pl.when(j + 1 < CH)
                def _():
                    pltpu.async_copy(g_hbm.at[srcv.at[j + 1]], buf_nxt,
                                     sem_nxt)

                pltpu.sync_copy(buf_cur, acc.at[dstv.at[j]], add=True)

            @pl.when(j % 2 == 0)
            def _():
                one(rowbuf0, sem0, rowbuf1, sem1)

            @pl.when(j % 2 == 1)
            def _():
                one(rowbuf1, sem1, rowbuf0, sem0)

            return carry2

        lax.fori_loop(0, CH, body, 0)
        return carry

    lax.fori_loop(0, NCHUNK, chunk, 0)
    plsc.subcore_barrier()

    @pl.when(sid < 15)
    def _():
        pltpu.sync_copy(acc.at[pl.ds(row0, RPT_A)],
                        a_hbm.at[cid, pl.ds(row0, RPT_A)])

    @pl.when(sid == 15)
    def _():
        pltpu.sync_copy(acc.at[pl.ds(15 * RPT_A, RPT_B)],
                        a_hbm.at[cid, pl.ds(15 * RPT_A, RPT_B)])


def _tc_pre_body(x_ref, emb_ref, wp1_ref, bp1_ref, wp2_ref, bp2_ref,
                 wc1_ref, dpt_ref, h0_ref, dinv_ref, g1_ref):
    table = jnp.dot(emb_ref[...], wp1_ref[...],
                    preferred_element_type=_f32) + bp1_ref[...]
    table = jnp.dot(_leaky(table), wp2_ref[...],
                    preferred_element_type=_f32) + bp2_ref[...]
    onehot = (lax.broadcasted_iota(jnp.int32, (N, D), 1)
              == x_ref[...]).astype(_f32)
    h0 = jnp.dot(onehot, table, preferred_element_type=_f32)
    deg = dpt_ref[:, 0:1] + dpt_ref[:, 1:2] + 1.0
    dinv = lax.rsqrt(deg)
    h0_ref[...] = h0
    dinv_ref[...] = dinv
    g1_ref[...] = jnp.dot(h0, wc1_ref[...], preferred_element_type=_f32) * dinv


_tc_pre = pl.pallas_call(
    _tc_pre_body,
    out_shape=[
        jax.ShapeDtypeStruct((N, D), _f32),
        jax.ShapeDtypeStruct((N, 1), _f32),
        jax.ShapeDtypeStruct((N, D), _f32),
    ],
)


def _tc_mid_body(a_ref, hprev_ref, dinv_ref, bc_ref, wcn_ref,
                 hnew_ref, gnext_ref):
    s = a_ref[0] + a_ref[1]
    hnew = _leaky(dinv_ref[...] * s + bc_ref[...]) + hprev_ref[...]
    hnew_ref[...] = hnew
    gnext_ref[...] = jnp.dot(hnew, wcn_ref[...],
                             preferred_element_type=_f32) * dinv_ref[...]


_tc_mid = pl.pallas_call(
    _tc_mid_body,
    out_shape=[
        jax.ShapeDtypeStruct((N, D), _f32),
        jax.ShapeDtypeStruct((N, D), _f32),
    ],
)


def _tc_post_body(a_ref, hprev_ref, dinv_ref, bc3_ref, wpost1_ref,
                  wpost2_ref, pred_ref):
    s = a_ref[0] + a_ref[1]
    h3 = _leaky(dinv_ref[...] * s + bc3_ref[...]) + hprev_ref[...]
    p = _leaky(jnp.dot(h3, wpost1_ref[...], preferred_element_type=_f32))
    pred_ref[...] = jnp.dot(p, wpost2_ref[...], preferred_element_type=_f32)


_tc_post = pl.pallas_call(
    _tc_post_body,
    out_shape=jax.ShapeDtypeStruct((N, 1), _f32),
)


def kernel(x, edge_index, emb, W_pre1, b_pre1, W_pre2, b_pre2,
           Wc1, bc1, Wc2, bc2, Wc3, bc3, W_post1, W_post2):
    ei = edge_index.astype(jnp.int32)
    src3 = ei[0].reshape(NW, NCHUNK, CH, WIN)
    dst3 = ei[1].reshape(NW, NCHUNK, CH, WIN)
    x2 = x.astype(jnp.int32).reshape(N, 1)
    zero_n = jnp.zeros((N,), _f32)
    zrows = jnp.zeros((RPT_B, D), _f32)

    dp = _sc_deg(dst3, zero_n)
    dpt = dp.T
    h0, dinv, g = _tc_pre(x2, emb, W_pre1, b_pre1.reshape(1, D),
                          W_pre2, b_pre2.reshape(1, D), Wc1, dpt)
    h = h0
    for bc, wnext in ((bc1, Wc2), (bc2, Wc3)):
        a = _sc_agg(g, src3, dst3, zrows)
        h, g = _tc_mid(a, h, dinv, bc.reshape(1, D), wnext)
    a = _sc_agg(g, src3, dst3, zrows)
    return _tc_post(a, h, dinv, bc3.reshape(1, D), W_post1, W_post2)


# speedup vs baseline: 26.5821x; 1.3696x over previous
import functools

import jax
import jax.numpy as jnp
from jax import lax
from jax.experimental import pallas as pl
from jax.experimental.pallas import tpu as pltpu
from jax.experimental.pallas import tpu_sc as plsc

N = 10000
D = 128
E = 320000
NC = 2
NS = 16
NW = NC * NS
EPW = E // NW
WIN = 80
NWIN = EPW // WIN
NCHUNK = 5
CH = NWIN // NCHUNK
RPT_A = 624
RPT_B = N - 15 * RPT_A

_f32 = jnp.float32


def _leaky(v):
    return jnp.where(v >= 0, v, 0.01 * v)


_sc_mesh = plsc.VectorSubcoreMesh(core_axis_name="c", subcore_axis_name="s")


@functools.partial(
    pl.kernel,
    mesh=_sc_mesh,
    out_type=jax.ShapeDtypeStruct((NC, N), _f32),
    scratch_types=[
        pltpu.VMEM_SHARED((N,), _f32),
        pltpu.VMEM((NCHUNK, CH, WIN), jnp.int32),
        pltpu.VMEM((WIN,), _f32),
    ],
)
def _sc_deg(dst_hbm, zero_hbm, dp_hbm, dacc, dstv, onesv):
    cid = lax.axis_index("c")
    sid = lax.axis_index("s")
    wid = sid * NC + cid
    pltpu.sync_copy(dst_hbm.at[wid], dstv)
    for k in range(WIN // 16):
        onesv[pl.ds(k * 16, 16)] = jnp.ones((16,), _f32)

    @pl.when(sid == 0)
    def _():
        pltpu.sync_copy(zero_hbm, dacc)

    plsc.subcore_barrier()

    def chunk(c, carry):
        def body(j, carry2):
            pltpu.sync_copy(onesv, dacc.at[dstv.at[c, j]], add=True)
            return carry2

        lax.fori_loop(0, CH, body, 0)
        return carry

    lax.fori_loop(0, NCHUNK, chunk, 0)
    plsc.subcore_barrier()

    @pl.when(sid == 0)
    def _():
        pltpu.sync_copy(dacc, dp_hbm.at[cid])


@functools.partial(
    pl.kernel,
    mesh=_sc_mesh,
    out_type=jax.ShapeDtypeStruct((NC, N, D), _f32),
    scratch_types=[
        pltpu.VMEM_SHARED((N, D), _f32),
        pltpu.VMEM((CH, WIN), jnp.int32),
        pltpu.VMEM((CH, WIN), jnp.int32),
        pltpu.VMEM((WIN, D), _f32),
        pltpu.VMEM((WIN, D), _f32),
        pltpu.VMEM((WIN, D), _f32),
        pltpu.SemaphoreType.DMA,
        pltpu.SemaphoreType.DMA,
        pltpu.SemaphoreType.DMA,
        pltpu.SemaphoreType.DMA,
    ],
)
def _sc_agg(g_hbm, src_hbm, dst_hbm, zrows_hbm, a_hbm, acc, srcv, dstv,
            rowbuf0, rowbuf1, rowbuf2, gsem0, gsem1, gsem2, ssem):
    cid = lax.axis_index("c")
    sid = lax.axis_index("s")
    wid = sid * NC + cid
    row0 = pl.multiple_of(sid * RPT_A, 8)

    @pl.when((cid == 0) & (sid < 15))
    def _():
        pltpu.sync_copy(g_hbm.at[pl.ds(row0, RPT_A)], acc.at[pl.ds(row0, RPT_A)])

    @pl.when((cid == 0) & (sid == 15))
    def _():
        pltpu.sync_copy(g_hbm.at[pl.ds(15 * RPT_A, RPT_B)],
                        acc.at[pl.ds(15 * RPT_A, RPT_B)])

    @pl.when((cid == 1) & (sid < 15))
    def _():
        pltpu.sync_copy(zrows_hbm.at[pl.ds(0, RPT_A)], acc.at[pl.ds(row0, RPT_A)])

    @pl.when((cid == 1) & (sid == 15))
    def _():
        pltpu.sync_copy(zrows_hbm, acc.at[pl.ds(15 * RPT_A, RPT_B)])

    plsc.subcore_barrier()

    bufs = (rowbuf0, rowbuf1, rowbuf2)
    gsems = (gsem0, gsem1, gsem2)
    dummy = g_hbm.at[pl.ds(0, WIN)]

    def chunk(c, carry):
        pltpu.sync_copy(src_hbm.at[wid, c], srcv)
        pltpu.sync_copy(dst_hbm.at[wid, c], dstv)
        pltpu.async_copy(g_hbm.at[srcv.at[0]], rowbuf0, gsem0)
        pltpu.async_copy(g_hbm.at[srcv.at[1]], rowbuf1, gsem1)

        def body(j, carry2):
            def one(k):
                bc, gc = bufs[k], gsems[k]
                bn, gn = bufs[(k + 2) % 3], gsems[(k + 2) % 3]
                pltpu.make_async_copy(dummy, bc, gc).wait()
                pltpu.async_copy(bc, acc.at[dstv.at[j]], ssem, add=True)

                @pl.when(j + 2 < CH)
                def _():
                    @pl.when(j >= 1)
                    def _():
                        pltpu.make_async_copy(dummy, rowbuf0, ssem).wait()

                    pltpu.async_copy(g_hbm.at[srcv.at[j + 2]], bn, gn)

            for k in range(3):
                @pl.when(j % 3 == k)
                def _(k=k):
                    one(k)

            return carry2

        lax.fori_loop(0, CH, body, 0)
        for _ in range(3):
            pltpu.make_async_copy(dummy, rowbuf0, ssem).wait()
        return carry

    lax.fori_loop(0, NCHUNK, chunk, 0)
    plsc.subcore_barrier()

    @pl.when(sid < 15)
    def _():
        pltpu.sync_copy(acc.at[pl.ds(row0, RPT_A)],
                        a_hbm.at[cid, pl.ds(row0, RPT_A)])

    @pl.when(sid == 15)
    def _():
        pltpu.sync_copy(acc.at[pl.ds(15 * RPT_A, RPT_B)],
                        a_hbm.at[cid, pl.ds(15 * RPT_A, RPT_B)])


def _tc_pre_body(x_ref, emb_ref, wp1_ref, bp1_ref, wp2_ref, bp2_ref,
                 wc1_ref, dpt_ref, h0_ref, dinv_ref, g1_ref):
    table = jnp.dot(emb_ref[...], wp1_ref[...],
                    preferred_element_type=_f32) + bp1_ref[...]
    table = jnp.dot(_leaky(table), wp2_ref[...],
                    preferred_element_type=_f32) + bp2_ref[...]
    onehot = (lax.broadcasted_iota(jnp.int32, (N, D), 1)
              == x_ref[...]).astype(_f32)
    h0 = jnp.dot(onehot, table, preferred_element_type=_f32)
    deg = dpt_ref[:, 0:1] + dpt_ref[:, 1:2] + 1.0
    dinv = lax.rsqrt(deg)
    h0_ref[...] = h0
    dinv_ref[...] = dinv
    g1_ref[...] = jnp.dot(h0, wc1_ref[...], preferred_element_type=_f32) * dinv


_tc_pre = pl.pallas_call(
    _tc_pre_body,
    out_shape=[
        jax.ShapeDtypeStruct((N, D), _f32),
        jax.ShapeDtypeStruct((N, 1), _f32),
        jax.ShapeDtypeStruct((N, D), _f32),
    ],
)


def _tc_mid_body(a_ref, hprev_ref, dinv_ref, bc_ref, wcn_ref,
                 hnew_ref, gnext_ref):
    s = a_ref[0] + a_ref[1]
    hnew = _leaky(dinv_ref[...] * s + bc_ref[...]) + hprev_ref[...]
    hnew_ref[...] = hnew
    gnext_ref[...] = jnp.dot(hnew, wcn_ref[...],
                             preferred_element_type=_f32) * dinv_ref[...]


_tc_mid = pl.pallas_call(
    _tc_mid_body,
    out_shape=[
        jax.ShapeDtypeStruct((N, D), _f32),
        jax.ShapeDtypeStruct((N, D), _f32),
    ],
)


def _tc_post_body(a_ref, hprev_ref, dinv_ref, bc3_ref, wpost1_ref,
                  wpost2_ref, pred_ref):
    s = a_ref[0] + a_ref[1]
    h3 = _leaky(dinv_ref[...] * s + bc3_ref[...]) + hprev_ref[...]
    p = _leaky(jnp.dot(h3, wpost1_ref[...], preferred_element_type=_f32))
    pred_ref[...] = jnp.dot(p, wpost2_ref[...], preferred_element_type=_f32)


_tc_post = pl.pallas_call(
    _tc_post_body,
    out_shape=jax.ShapeDtypeStruct((N, 1), _f32),
)


def kernel(x, edge_index, emb, W_pre1, b_pre1, W_pre2, b_pre2,
           Wc1, bc1, Wc2, bc2, Wc3, bc3, W_post1, W_post2):
    ei = edge_index.astype(jnp.int32)
    src3 = ei[0].reshape(NW, NCHUNK, CH, WIN)
    dst3 = ei[1].reshape(NW, NCHUNK, CH, WIN)
    x2 = x.astype(jnp.int32).reshape(N, 1)
    zero_n = jnp.zeros((N,), _f32)
    zrows = jnp.zeros((RPT_B, D), _f32)

    dp = _sc_deg(dst3, zero_n)
    dpt = dp.T
    h0, dinv, g = _tc_pre(x2, emb, W_pre1, b_pre1.reshape(1, D),
                          W_pre2, b_pre2.reshape(1, D), Wc1, dpt)
    h = h0
    for bc, wnext in ((bc1, Wc2), (bc2, Wc3)):
        a = _sc_agg(g, src3, dst3, zrows)
        h, g = _tc_mid(a, h, dinv, bc.reshape(1, D), wnext)
    a = _sc_agg(g, src3, dst3, zrows)
    return _tc_post(a, h, dinv, bc3.reshape(1, D), W_post1, W_post2)


# speedup vs baseline: 26.7627x; 1.0068x over previous
import functools

import jax
import jax.numpy as jnp
from jax import lax
from jax.experimental import pallas as pl
from jax.experimental.pallas import tpu as pltpu
from jax.experimental.pallas import tpu_sc as plsc

N = 10000
D = 128
E = 320000
NC = 2
NS = 16
NW = NC * NS
EPW = E // NW
WIN = 80
NWIN = EPW // WIN
NCHUNK = 5
CH = NWIN // NCHUNK
RPT_A = 624
RPT_B = N - 15 * RPT_A

_f32 = jnp.float32


def _leaky(v):
    return jnp.where(v >= 0, v, 0.01 * v)


_sc_mesh = plsc.VectorSubcoreMesh(core_axis_name="c", subcore_axis_name="s")


@functools.partial(
    pl.kernel,
    mesh=_sc_mesh,
    out_type=jax.ShapeDtypeStruct((NC, N), _f32),
    scratch_types=[
        pltpu.VMEM_SHARED((N,), _f32),
        pltpu.VMEM((NCHUNK, CH, WIN), jnp.int32),
        pltpu.VMEM((WIN,), _f32),
        pltpu.SemaphoreType.DMA,
    ],
)
def _sc_deg(dst_hbm, zero_hbm, dp_hbm, dacc, dstv, onesv, ssem):
    cid = lax.axis_index("c")
    sid = lax.axis_index("s")
    wid = sid * NC + cid
    pltpu.sync_copy(dst_hbm.at[wid], dstv)
    for k in range(WIN // 16):
        onesv[pl.ds(k * 16, 16)] = jnp.ones((16,), _f32)

    @pl.when(sid == 0)
    def _():
        pltpu.sync_copy(zero_hbm, dacc)

    plsc.subcore_barrier()

    def chunk(c, carry):
        def body(j, carry2):
            pltpu.async_copy(onesv, dacc.at[dstv.at[c, j]], ssem, add=True)
            return carry2

        lax.fori_loop(0, CH, body, 0)

        def drain(j, carry2):
            pltpu.make_async_copy(zero_hbm.at[pl.ds(0, WIN)], onesv,
                                  ssem).wait()
            return carry2

        lax.fori_loop(0, CH, drain, 0)
        return carry

    lax.fori_loop(0, NCHUNK, chunk, 0)
    plsc.subcore_barrier()

    @pl.when(sid == 0)
    def _():
        pltpu.sync_copy(dacc, dp_hbm.at[cid])


@functools.partial(
    pl.kernel,
    mesh=_sc_mesh,
    out_type=jax.ShapeDtypeStruct((NC, N, D), _f32),
    scratch_types=[
        pltpu.VMEM_SHARED((N, D), _f32),
        pltpu.VMEM((CH, WIN), jnp.int32),
        pltpu.VMEM((CH, WIN), jnp.int32),
        pltpu.VMEM((WIN, D), _f32),
        pltpu.VMEM((WIN, D), _f32),
        pltpu.VMEM((WIN, D), _f32),
        pltpu.SemaphoreType.DMA,
        pltpu.SemaphoreType.DMA,
        pltpu.SemaphoreType.DMA,
        pltpu.SemaphoreType.DMA,
    ],
)
def _sc_agg(g_hbm, src_hbm, dst_hbm, zrows_hbm, a_hbm, acc, srcv, dstv,
            rowbuf0, rowbuf1, rowbuf2, gsem0, gsem1, gsem2, ssem):
    cid = lax.axis_index("c")
    sid = lax.axis_index("s")
    wid = sid * NC + cid
    row0 = pl.multiple_of(sid * RPT_A, 8)

    @pl.when((cid == 0) & (sid < 15))
    def _():
        pltpu.sync_copy(g_hbm.at[pl.ds(row0, RPT_A)], acc.at[pl.ds(row0, RPT_A)])

    @pl.when((cid == 0) & (sid == 15))
    def _():
        pltpu.sync_copy(g_hbm.at[pl.ds(15 * RPT_A, RPT_B)],
                        acc.at[pl.ds(15 * RPT_A, RPT_B)])

    @pl.when((cid == 1) & (sid < 15))
    def _():
        pltpu.sync_copy(zrows_hbm.at[pl.ds(0, RPT_A)], acc.at[pl.ds(row0, RPT_A)])

    @pl.when((cid == 1) & (sid == 15))
    def _():
        pltpu.sync_copy(zrows_hbm, acc.at[pl.ds(15 * RPT_A, RPT_B)])

    plsc.subcore_barrier()

    bufs = (rowbuf0, rowbuf1, rowbuf2)
    gsems = (gsem0, gsem1, gsem2)
    dummy = g_hbm.at[pl.ds(0, WIN)]

    def chunk(c, carry):
        pltpu.sync_copy(src_hbm.at[wid, c], srcv)
        pltpu.sync_copy(dst_hbm.at[wid, c], dstv)
        pltpu.async_copy(g_hbm.at[srcv.at[0]], rowbuf0, gsem0)
        pltpu.async_copy(g_hbm.at[srcv.at[1]], rowbuf1, gsem1)

        def body(j, carry2):
            def one(k):
                bc, gc = bufs[k], gsems[k]
                bn, gn = bufs[(k + 2) % 3], gsems[(k + 2) % 3]
                pltpu.make_async_copy(dummy, bc, gc).wait()
                pltpu.async_copy(bc, acc.at[dstv.at[j]], ssem, add=True)

                @pl.when(j + 2 < CH)
                def _():
                    @pl.when(j >= 1)
                    def _():
                        pltpu.make_async_copy(dummy, rowbuf0, ssem).wait()

                    pltpu.async_copy(g_hbm.at[srcv.at[j + 2]], bn, gn)

            for k in range(3):
                @pl.when(j % 3 == k)
                def _(k=k):
                    one(k)

            return carry2

        lax.fori_loop(0, CH, body, 0)
        for _ in range(3):
            pltpu.make_async_copy(dummy, rowbuf0, ssem).wait()
        return carry

    lax.fori_loop(0, NCHUNK, chunk, 0)
    plsc.subcore_barrier()

    @pl.when(sid < 15)
    def _():
        pltpu.sync_copy(acc.at[pl.ds(row0, RPT_A)],
                        a_hbm.at[cid, pl.ds(row0, RPT_A)])

    @pl.when(sid == 15)
    def _():
        pltpu.sync_copy(acc.at[pl.ds(15 * RPT_A, RPT_B)],
                        a_hbm.at[cid, pl.ds(15 * RPT_A, RPT_B)])


def _tc_emb_body(x_ref, emb_ref, wp1_ref, bp1_ref, wp2_ref, bp2_ref, h0_ref):
    table = jnp.dot(emb_ref[...], wp1_ref[...],
                    preferred_element_type=_f32) + bp1_ref[...]
    table = jnp.dot(_leaky(table), wp2_ref[...],
                    preferred_element_type=_f32) + bp2_ref[...]
    onehot = (lax.broadcasted_iota(jnp.int32, (N, D), 1)
              == x_ref[...]).astype(_f32)
    h0_ref[...] = jnp.dot(onehot, table, preferred_element_type=_f32)


_tc_emb = pl.pallas_call(
    _tc_emb_body,
    out_shape=jax.ShapeDtypeStruct((N, D), _f32),
)


def _tc_scale_body(h0_ref, dpt_ref, wc1_ref, dinv_ref, g1_ref):
    deg = dpt_ref[:, 0:1] + dpt_ref[:, 1:2] + 1.0
    dinv = lax.rsqrt(deg)
    dinv_ref[...] = dinv
    g1_ref[...] = jnp.dot(h0_ref[...], wc1_ref[...],
                          preferred_element_type=_f32) * dinv


_tc_scale = pl.pallas_call(
    _tc_scale_body,
    out_shape=[
        jax.ShapeDtypeStruct((N, 1), _f32),
        jax.ShapeDtypeStruct((N, D), _f32),
    ],
)


def _tc_mid_body(a_ref, hprev_ref, dinv_ref, bc_ref, wcn_ref,
                 hnew_ref, gnext_ref):
    s = a_ref[0] + a_ref[1]
    hnew = _leaky(dinv_ref[...] * s + bc_ref[...]) + hprev_ref[...]
    hnew_ref[...] = hnew
    gnext_ref[...] = jnp.dot(hnew, wcn_ref[...],
                             preferred_element_type=_f32) * dinv_ref[...]


_tc_mid = pl.pallas_call(
    _tc_mid_body,
    out_shape=[
        jax.ShapeDtypeStruct((N, D), _f32),
        jax.ShapeDtypeStruct((N, D), _f32),
    ],
)


def _tc_post_body(a_ref, hprev_ref, dinv_ref, bc3_ref, wpost1_ref,
                  wpost2_ref, pred_ref):
    s = a_ref[0] + a_ref[1]
    h3 = _leaky(dinv_ref[...] * s + bc3_ref[...]) + hprev_ref[...]
    p = _leaky(jnp.dot(h3, wpost1_ref[...], preferred_element_type=_f32))
    pred_ref[...] = jnp.dot(p, wpost2_ref[...], preferred_element_type=_f32)


_tc_post = pl.pallas_call(
    _tc_post_body,
    out_shape=jax.ShapeDtypeStruct((N, 1), _f32),
)


def kernel(x, edge_index, emb, W_pre1, b_pre1, W_pre2, b_pre2,
           Wc1, bc1, Wc2, bc2, Wc3, bc3, W_post1, W_post2):
    ei = edge_index.astype(jnp.int32)
    src3 = ei[0].reshape(NW, NCHUNK, CH, WIN)
    dst3 = ei[1].reshape(NW, NCHUNK, CH, WIN)
    x2 = x.astype(jnp.int32).reshape(N, 1)
    zero_n = jnp.zeros((N,), _f32)
    zrows = jnp.zeros((RPT_B, D), _f32)

    dp = _sc_deg(dst3, zero_n)
    dpt = dp.T
    h = _tc_emb(x2, emb, W_pre1, b_pre1.reshape(1, D),
                W_pre2, b_pre2.reshape(1, D))
    dinv, g = _tc_scale(h, dpt, Wc1)
    for bc, wnext in ((bc1, Wc2), (bc2, Wc3)):
        a = _sc_agg(g, src3, dst3, zrows)
        h, g = _tc_mid(a, h, dinv, bc.reshape(1, D), wnext)
    a = _sc_agg(g, src3, dst3, zrows)
    return _tc_post(a, h, dinv, bc3.reshape(1, D), W_post1, W_post2)
